# baseline (speedup 1.0000x reference)
import functools
import math

import jax
import jax.numpy as jnp
from jax import lax
from jax.experimental import pallas as pl
from jax.experimental.pallas import tpu as pltpu
from jax.experimental.pallas import tpu_sc as plsc

VOCAB = 1000000
EMB = 64
B = 4096
L = 200
SCALE = math.sqrt(EMB)

_info = plsc.get_sparse_core_info()
NC, NS, LANES = _info.num_cores, _info.num_subcores, _info.num_lanes
NW = NC * NS
ROWS_W = B // NW
NB = 4
GROUPS = ROWS_W // NB
RU = 8


def _body(tok_hbm, table_hbm, out_hbm, idx_v, buf, gsems, wsems):
    wid = lax.axis_index("s") * NC + lax.axis_index("c")
    w_base = wid * ROWS_W

    def gather_start(g, b):
        pltpu.async_copy(table_hbm.at[idx_v.at[g]], buf.at[b], gsems[b])

    def gather_wait(g, b):
        pltpu.make_async_copy(table_hbm.at[idx_v.at[g]], buf.at[b], gsems[b]).wait()

    def wb_start(g, b):
        pltpu.async_copy(buf.at[b], out_hbm.at[w_base + g], wsems[b])

    def wb_wait(g, b):
        pltpu.make_async_copy(buf.at[b], out_hbm.at[w_base + g], wsems[b]).wait()

    def scale(b):
        def srow(r0, c):
            for r in range(RU):
                row = r0 * RU + r
                for j in range(EMB // LANES):
                    sl = pl.ds(j * LANES, LANES)
                    buf[b, row, sl] = buf[b, row, sl] * SCALE
            return c

        lax.fori_loop(0, L // RU, srow, 0)

    pltpu.sync_copy(tok_hbm.at[pl.ds(w_base, ROWS_W)], idx_v)
    for b in range(NB):
        gather_start(b, b)

    for b in range(NB):
        gather_wait(b, b)
        scale(b)
        wb_start(b, b)

    def group(go, carry):
        for b in range(NB):
            g = go * NB + b
            wb_wait(g - NB, b)
            gather_start(g, b)
        for b in range(NB):
            g = go * NB + b
            gather_wait(g, b)
            scale(b)
            wb_start(g, b)
        return carry

    lax.fori_loop(1, GROUPS, group, 0)

    for b in range(NB):
        wb_wait((GROUPS - 1) * NB + b, b)


@functools.partial(jax.jit, static_argnames=())
def kernel(tokens, table):
    mesh = plsc.VectorSubcoreMesh(core_axis_name="c", subcore_axis_name="s")
    run = pl.kernel(
        _body,
        out_type=jax.ShapeDtypeStruct((B, L, EMB), jnp.float32),
        mesh=mesh,
        scratch_types=[
            pltpu.VMEM((ROWS_W, L), jnp.int32),
            pltpu.VMEM((NB, L, EMB), jnp.float32),
            [pltpu.SemaphoreType.DMA] * NB,
            [pltpu.SemaphoreType.DMA] * NB,
        ],
        compiler_params=pltpu.CompilerParams(use_tc_tiling_on_sc=False),
    )
    return run(tokens.astype(jnp.int32), table)
